# baseline (speedup 1.0000x reference)
import functools

import jax
import jax.numpy as jnp
from jax import lax
from jax.experimental import pallas as pl
from jax.experimental.pallas import tpu as pltpu
from jax.experimental.pallas import tpu_sc as plsc

_NEG = float(jnp.finfo(jnp.float32).min)
_F32 = jnp.float32

_NC, _NS = 2, 16
_NW = _NC * _NS


def _gelu(x):
    return 0.5 * x * (1.0 + lax.erf(x * (2.0 ** -0.5)))


def _prologue_body(qe_ref, qm_ref, ee_ref, wproj_ref, bproj_ref, wm0b_ref,
                   q_ref, ent0_ref, entw0_ref):
    j = pl.program_id(1)
    ent0 = (jnp.dot(ee_ref[0], wproj_ref[...], preferred_element_type=_F32)
            + bproj_ref[...])
    ent0_ref[0] = ent0
    entw0_ref[0] = jnp.dot(ent0, wm0b_ref[...], preferred_element_type=_F32)

    @pl.when(j == 0)
    def _():
        qe = (jnp.dot(qe_ref[0], wproj_ref[...], preferred_element_type=_F32)
              + bproj_ref[...])
        qm = qm_ref[0]
        num = jnp.sum(qe * qm, axis=0)
        den = jnp.sum(qm, axis=0)
        q_ref[0, 0] = num / den


def _prologue(question_embedding, qm_b, entity_embedding,
              W_proj, b_proj, wm0b, TN):
    B, LQ, E = question_embedding.shape
    _, N, _ = entity_embedding.shape
    H = W_proj.shape[1]
    NB = N // TN
    return pl.pallas_call(
        _prologue_body,
        grid=(B, NB),
        in_specs=[
            pl.BlockSpec((1, LQ, E), lambda b, j: (b, 0, 0)),
            pl.BlockSpec((1, LQ, H), lambda b, j: (b, 0, 0)),
            pl.BlockSpec((1, TN, E), lambda b, j: (b, j, 0)),
            pl.BlockSpec((E, H), lambda b, j: (0, 0)),
            pl.BlockSpec((1, H), lambda b, j: (0, 0)),
            pl.BlockSpec((H, H), lambda b, j: (0, 0)),
        ],
        out_specs=[
            pl.BlockSpec((1, 1, H), lambda b, j: (b, 0, 0)),
            pl.BlockSpec((1, TN, H), lambda b, j: (b, j, 0)),
            pl.BlockSpec((1, TN, H), lambda b, j: (b, j, 0)),
        ],
        out_shape=[
            jax.ShapeDtypeStruct((B, 1, H), _F32),
            jax.ShapeDtypeStruct((B, N, H), _F32),
            jax.ShapeDtypeStruct((B, N, H), _F32),
        ],
        compiler_params=pltpu.CompilerParams(
            dimension_semantics=("parallel", "arbitrary")),
    )(question_embedding, qm_b, entity_embedding, W_proj, b_proj, wm0b)


def _hop0_body(rel_ref, tail_ref, me_ref, mo_ref, ent_ref, q_ref,
               wrsq_ref, crs_ref, AX_ref, c2_ref,
               wut_ref, wub_ref, bu_ref, wnext_ref,
               wle_ref, wlo_ref, we_ref, wo_ref, ent1_ref, entw1_ref):
    TN, P, D2 = rel_ref.shape[1], rel_ref.shape[2], rel_ref.shape[3]
    H = wut_ref.shape[0]
    relF = rel_ref[0].reshape(TN * P, D2)
    O = jnp.dot(relF, AX_ref[...], preferred_element_type=_F32)
    qdot = jnp.sum(q_ref[0] * wrsq_ref[...]) + crs_ref[0, 0]
    me = me_ref[0][:, :, None]
    mo = mo_ref[0][:, :, None]
    wle = O[:, D2:D2 + 1].reshape(TN, P, 1) + qdot + (1.0 - me) * _NEG
    wlo = O[:, D2 + 1:D2 + 2].reshape(TN, P, 1) + qdot + (1.0 - mo) * _NEG
    wle_ref[0] = wle[..., 0]
    wlo_ref[0] = wlo[..., 0]
    m = jnp.max(jnp.maximum(wle, wlo), axis=1, keepdims=True)
    ee = jnp.exp(wle - m)
    eo = jnp.exp(wlo - m)
    s = jnp.sum(ee + eo, axis=1, keepdims=True)
    we3 = me * (ee / s)
    wo3 = mo * (eo / s)
    we_ref[0] = we3[..., 0]
    wo_ref[0] = wo3[..., 0]

    pre = O[:, :D2] + tail_ref[0].reshape(TN * P, D2) + c2_ref[...]
    msg = _gelu(pre).reshape(TN, P, D2)
    w2 = jnp.concatenate([jnp.broadcast_to(we3, (TN, P, H)),
                          jnp.broadcast_to(wo3, (TN, P, H))], axis=-1)
    agg2 = jnp.sum(msg * w2, axis=1)
    agg = agg2[:, :H] + agg2[:, H:]
    u = (jnp.dot(ent_ref[0], wut_ref[...], preferred_element_type=_F32)
         + jnp.dot(agg, wub_ref[...], preferred_element_type=_F32)
         + bu_ref[...])
    ent1 = _gelu(u)
    ent1_ref[0] = ent1
    entw1_ref[0] = jnp.dot(ent1, wnext_ref[...], preferred_element_type=_F32)


def _hop0(rel2, tail2, mask_e, mask_o, ent0, q,
          wrsq, crs, AX0, c2, wut, wub, bu, wnext, TN):
    B, N, P, D2 = rel2.shape
    H = wut.shape[0]
    NB = N // TN
    NC = AX0.shape[1]
    wfull = lambda *s: pl.BlockSpec(s, lambda b, j: tuple(0 for _ in s))
    return pl.pallas_call(
        _hop0_body,
        grid=(B, NB),
        in_specs=[
            pl.BlockSpec((1, TN, P, D2), lambda b, j: (b, j, 0, 0)),
            pl.BlockSpec((1, TN, P, D2), lambda b, j: (b, j, 0, 0)),
            pl.BlockSpec((1, TN, P), lambda b, j: (b, j, 0)),
            pl.BlockSpec((1, TN, P), lambda b, j: (b, j, 0)),
            pl.BlockSpec((1, TN, H), lambda b, j: (b, j, 0)),
            pl.BlockSpec((1, 1, H), lambda b, j: (b, 0, 0)),
            wfull(1, H), wfull(1, 1), wfull(D2, NC), wfull(1, D2),
            wfull(H, H), wfull(H, H), wfull(1, H), wfull(H, H),
        ],
        out_specs=[
            pl.BlockSpec((1, TN, P), lambda b, j: (b, j, 0)),
            pl.BlockSpec((1, TN, P), lambda b, j: (b, j, 0)),
            pl.BlockSpec((1, TN, P), lambda b, j: (b, j, 0)),
            pl.BlockSpec((1, TN, P), lambda b, j: (b, j, 0)),
            pl.BlockSpec((1, TN, H), lambda b, j: (b, j, 0)),
            pl.BlockSpec((1, TN, H), lambda b, j: (b, j, 0)),
        ],
        out_shape=[
            jax.ShapeDtypeStruct((B, N, P), _F32),
            jax.ShapeDtypeStruct((B, N, P), _F32),
            jax.ShapeDtypeStruct((B, N, P), _F32),
            jax.ShapeDtypeStruct((B, N, P), _F32),
            jax.ShapeDtypeStruct((B, N, H), _F32),
            jax.ShapeDtypeStruct((B, N, H), _F32),
        ],
        compiler_params=pltpu.CompilerParams(
            dimension_semantics=("parallel", "parallel")),
    )(rel2, tail2, mask_e, mask_o, ent0, q,
      wrsq, crs, AX0, c2, wut, wub, bu, wnext)


def _hop1_body(rel_ref, tail_ref, we_ref, wo_ref, ent_ref,
               A2_ref, c2_ref, wut_ref, wub_ref, bu_ref,
               ent2_ref):
    TN, P, D2 = rel_ref.shape[1], rel_ref.shape[2], rel_ref.shape[3]
    H = wut_ref.shape[0]
    relF = rel_ref[0].reshape(TN * P, D2)
    M2 = jnp.dot(relF, A2_ref[...], preferred_element_type=_F32)
    pre = M2 + tail_ref[0].reshape(TN * P, D2) + c2_ref[...]
    msg = _gelu(pre).reshape(TN, P, D2)
    we3 = we_ref[0][:, :, None]
    wo3 = wo_ref[0][:, :, None]
    w2 = jnp.concatenate([jnp.broadcast_to(we3, (TN, P, H)),
                          jnp.broadcast_to(wo3, (TN, P, H))], axis=-1)
    agg2 = jnp.sum(msg * w2, axis=1)
    agg = agg2[:, :H] + agg2[:, H:]
    u = (jnp.dot(ent_ref[0], wut_ref[...], preferred_element_type=_F32)
         + jnp.dot(agg, wub_ref[...], preferred_element_type=_F32)
         + bu_ref[...])
    ent2_ref[0] = _gelu(u)


def _hop1(rel2, tail2, w_e, w_o, ent1, A2, c2, wut, wub, bu, TN):
    B, N, P, D2 = rel2.shape
    H = wut.shape[0]
    NB = N // TN
    wfull = lambda *s: pl.BlockSpec(s, lambda b, j: tuple(0 for _ in s))
    return pl.pallas_call(
        _hop1_body,
        grid=(B, NB),
        in_specs=[
            pl.BlockSpec((1, TN, P, D2), lambda b, j: (b, j, 0, 0)),
            pl.BlockSpec((1, TN, P, D2), lambda b, j: (b, j, 0, 0)),
            pl.BlockSpec((1, TN, P), lambda b, j: (b, j, 0)),
            pl.BlockSpec((1, TN, P), lambda b, j: (b, j, 0)),
            pl.BlockSpec((1, TN, H), lambda b, j: (b, j, 0)),
            wfull(D2, D2), wfull(1, D2), wfull(H, H), wfull(H, H),
            wfull(1, H),
        ],
        out_specs=[pl.BlockSpec((1, TN, H), lambda b, j: (b, j, 0))],
        out_shape=[jax.ShapeDtypeStruct((B, N, H), _F32)],
        compiler_params=pltpu.CompilerParams(
            dimension_semantics=("parallel", "parallel")),
    )(rel2, tail2, w_e, w_o, ent1, A2, c2, wut, wub, bu)[0]


def _epilogue_body(ent2_ref, q_ref, nmask_ref, ws_ref, bs_ref,
                   score_ref, eq_ref):
    ent2 = ent2_ref[0]
    score = (jnp.sum(ent2 * ws_ref[...], axis=-1) + bs_ref[0, 0]
             + (1.0 - nmask_ref[0, 0]) * _NEG)
    score_ref[0, 0] = score
    eq_ref[0, 0] = jnp.sum(ent2 * q_ref[0], axis=-1)


def _epilogue(ent2, q, node_mask, ws_row, bs):
    B, N, H = ent2.shape
    wfull = lambda *s: pl.BlockSpec(s, lambda b: tuple(0 for _ in s))
    score, eq = pl.pallas_call(
        _epilogue_body,
        grid=(B,),
        in_specs=[
            pl.BlockSpec((1, N, H), lambda b: (b, 0, 0)),
            pl.BlockSpec((1, 1, H), lambda b: (b, 0, 0)),
            pl.BlockSpec((1, 1, N), lambda b: (b, 0, 0)),
            wfull(1, H), wfull(1, 1),
        ],
        out_specs=[
            pl.BlockSpec((1, 1, N), lambda b: (b, 0, 0)),
            pl.BlockSpec((1, 1, N), lambda b: (b, 0, 0)),
        ],
        out_shape=[
            jax.ShapeDtypeStruct((B, 1, N), _F32),
            jax.ShapeDtypeStruct((B, 1, N), _F32),
        ],
    )(ent2, q, node_mask.reshape(B, 1, N), ws_row, bs)
    return score.reshape(B, N), eq.reshape(B, N)


def _sc_gather(table, idx, n_batches):
    T, H = table.shape
    E = idx.shape[0]
    per_w = E // _NW
    C = 800
    n_chunks = per_w // C
    n_per_batch = T // n_batches
    w_per_batch = _NW // n_batches
    mesh = plsc.VectorSubcoreMesh(core_axis_name="c", subcore_axis_name="s")

    @functools.partial(
        pl.kernel, mesh=mesh,
        out_type=jax.ShapeDtypeStruct((E, H), _F32),
        scratch_types=[
            pltpu.VMEM((per_w,), jnp.int32),
            pltpu.VMEM((C, H), _F32),
            pltpu.SemaphoreType.DMA,
        ],
        compiler_params=pltpu.CompilerParams(use_tc_tiling_on_sc=False),
    )
    def k(table_hbm, idx_hbm, out_hbm, idx_v, rows_v, sem):
        wid = lax.axis_index("s") * _NC + lax.axis_index("c")
        base = wid * per_w
        boff = (wid // w_per_batch) * n_per_batch
        pltpu.sync_copy(idx_hbm.at[pl.ds(base, per_w)], idx_v)

        def add_off(i):
            sl = pl.ds(i * 16, 16)
            idx_v[sl] = idx_v[sl] + boff
        pl.loop(0, per_w // 16)(add_off)

        def chunk(c):
            pltpu.async_copy(
                table_hbm.at[idx_v.at[pl.ds(c * C, C)]], rows_v, sem).wait()
            pltpu.sync_copy(rows_v, out_hbm.at[pl.ds(base + c * C, C)])
        pl.loop(0, n_chunks)(chunk)

    return k(table, idx)


def _sc_triple(eq_flat, adj_flat, wl_flat, n_batches, n_nodes, log2_r):
    E = adj_flat.shape[0]
    per_w = E // _NW
    C = 2000
    n_chunks = per_w // C
    w_per_batch = _NW // n_batches
    per_batch_e = per_w * w_per_batch
    mesh = plsc.VectorSubcoreMesh(core_axis_name="c", subcore_axis_name="s")

    @functools.partial(
        pl.kernel, mesh=mesh,
        out_type=jax.ShapeDtypeStruct((E,), _F32),
        scratch_types=[
            pltpu.VMEM((n_nodes,), _F32),
            pltpu.VMEM((C,), jnp.int32),
            pltpu.VMEM((C,), _F32),
            pltpu.VMEM((C,), _F32),
        ],
        compiler_params=pltpu.CompilerParams(needs_layout_passes=False),
    )
    def k(eq_hbm, adj_hbm, wl_hbm, out_hbm, eq_v, idx_v, wl_v, out_v):
        wid = lax.axis_index("s") * _NC + lax.axis_index("c")
        b = wid // w_per_batch
        gbase = wid * per_w
        lbase = gbase - b * per_batch_e
        pltpu.sync_copy(eq_hbm.at[pl.ds(b * n_nodes, n_nodes)], eq_v)

        def chunk(c):
            pltpu.sync_copy(adj_hbm.at[pl.ds(gbase + c * C, C)], idx_v)
            pltpu.sync_copy(wl_hbm.at[pl.ds(gbase + c * C, C)], wl_v)

            def step(i):
                sl = pl.ds(i * 16, 16)
                lane_e = lbase + c * C + i * 16 + lax.iota(jnp.int32, 16)
                src = lax.shift_right_logical(lane_e, log2_r)
                vsrc = plsc.load_gather(eq_v, [src])
                vtgt = plsc.load_gather(eq_v, [idx_v[sl]])
                out_v[sl] = vsrc + vtgt + wl_v[sl]
            pl.loop(0, C // 16)(step)
            pltpu.sync_copy(out_v, out_hbm.at[pl.ds(gbase + c * C, C)])
        pl.loop(0, n_chunks)(chunk)

    return k(eq_flat, adj_flat, wl_flat)


def kernel(question_embedding, question_mask, entity_embedding, rel_embedding,
           adj, node_mask, adj_mask,
           W_proj, b_proj, W_relp, b_relp,
           W_msg0, b_msg0, W_msg1, b_msg1,
           W_upd0, b_upd0, W_upd1, b_upd1,
           W_rs, b_rs, W_s, b_s):
    B, LQ, E = question_embedding.shape
    _, N, R, RD = rel_embedding.shape
    H = W_proj.shape[1]
    TN = 400
    log2_r = R.bit_length() - 1
    assert (1 << log2_r) == R

    P = R // 2
    D2 = 2 * RD
    A0 = W_relp @ W_msg0[:H]
    c0 = b_relp @ W_msg0[:H] + b_msg0
    A1 = W_relp @ W_msg1[:H]
    c1 = b_relp @ W_msg1[:H] + b_msg1
    ars = W_relp @ W_rs[:H, 0]
    crs = (b_relp @ W_rs[:H, 0] + b_rs[0]).reshape(1, 1)
    wrsq = W_rs[H:, 0][None]
    wm0b = W_msg0[H:]
    wm1b = W_msg1[H:]
    bproj = b_proj[None]
    bu0 = b_upd0[None]
    bu1 = b_upd1[None]
    ws_row = W_s[:, 0][None]
    bs = b_s.reshape(1, 1)
    qm_b = jnp.broadcast_to(question_mask[:, :, None], (B, LQ, H))

    z = jnp.zeros((RD, H), _F32)
    A2_0 = jnp.block([[A0, z], [z, A0]])
    A2_1 = jnp.block([[A1, z], [z, A1]])
    zc = jnp.zeros((RD,), _F32)
    AX0 = jnp.concatenate(
        [A2_0, jnp.concatenate([ars, zc])[:, None],
         jnp.concatenate([zc, ars])[:, None]], axis=1)
    AX0 = AX0.astype(jnp.bfloat16)
    A2_1 = A2_1.astype(jnp.bfloat16)
    c2_0 = jnp.concatenate([c0, c0])[None]
    c2_1 = jnp.concatenate([c1, c1])[None]

    rel2 = rel_embedding.astype(jnp.bfloat16).reshape(B, N, P, D2)
    mask_e = adj_mask[:, :, 0::2]
    mask_o = adj_mask[:, :, 1::2]
    adj_flat = adj.reshape(B * N * R)

    q, ent0, entw0 = _prologue(question_embedding, qm_b, entity_embedding,
                               W_proj, bproj, wm0b, TN)
    tail0 = _sc_gather(entw0.reshape(B * N, H), adj_flat, B)
    wl_e, wl_o, w_e, w_o, ent1, entw1 = _hop0(
        rel2, tail0.reshape(B, N, P, D2), mask_e, mask_o, ent0, q,
        wrsq, crs, AX0, c2_0, W_upd0[:H], W_upd0[H:], bu0, wm1b, TN)
    wl = jnp.stack([wl_e, wl_o], axis=-1).reshape(B, N, R)
    tail1 = _sc_gather(entw1.reshape(B * N, H), adj_flat, B)
    ent2 = _hop1(rel2, tail1.reshape(B, N, P, D2), w_e, w_o, ent1,
                 A2_1, c2_1, W_upd1[:H], W_upd1[H:], bu1, TN)
    score, eq = _epilogue(ent2, q, node_mask, ws_row, bs)
    ts = _sc_triple(eq.reshape(B * N), adj_flat, wl.reshape(B * N * R),
                    B, N, log2_r)
    return (ent2, score, wl, ts.reshape(B, N * R))


# speedup vs baseline: 1.2179x; 1.2179x over previous
import functools

import jax
import jax.numpy as jnp
from jax import lax
from jax.experimental import pallas as pl
from jax.experimental.pallas import tpu as pltpu
from jax.experimental.pallas import tpu_sc as plsc

_NEG = float(jnp.finfo(jnp.float32).min)
_F32 = jnp.float32

_NC, _NS = 2, 16
_NW = _NC * _NS


def _gelu(x):
    return 0.5 * x * (1.0 + lax.erf(x * (2.0 ** -0.5)))


def _prologue_body(qe_ref, qm_ref, ee_ref, wproj_ref, bproj_ref, wm0b_ref,
                   q_ref, ent0_ref, entw0_ref):
    j = pl.program_id(1)
    ent0 = (jnp.dot(ee_ref[0], wproj_ref[...], preferred_element_type=_F32)
            + bproj_ref[...])
    ent0_ref[0] = ent0
    entw0_ref[0] = jnp.dot(ent0, wm0b_ref[...], preferred_element_type=_F32)

    @pl.when(j == 0)
    def _():
        qe = (jnp.dot(qe_ref[0], wproj_ref[...], preferred_element_type=_F32)
              + bproj_ref[...])
        qm = qm_ref[0]
        num = jnp.sum(qe * qm, axis=0)
        den = jnp.sum(qm, axis=0)
        q_ref[0, 0] = num / den


def _prologue(question_embedding, qm_b, entity_embedding,
              W_proj, b_proj, wm0b, TN):
    B, LQ, E = question_embedding.shape
    _, N, _ = entity_embedding.shape
    H = W_proj.shape[1]
    NB = N // TN
    return pl.pallas_call(
        _prologue_body,
        grid=(B, NB),
        in_specs=[
            pl.BlockSpec((1, LQ, E), lambda b, j: (b, 0, 0)),
            pl.BlockSpec((1, LQ, H), lambda b, j: (b, 0, 0)),
            pl.BlockSpec((1, TN, E), lambda b, j: (b, j, 0)),
            pl.BlockSpec((E, H), lambda b, j: (0, 0)),
            pl.BlockSpec((1, H), lambda b, j: (0, 0)),
            pl.BlockSpec((H, H), lambda b, j: (0, 0)),
        ],
        out_specs=[
            pl.BlockSpec((1, 1, H), lambda b, j: (b, 0, 0)),
            pl.BlockSpec((1, TN, H), lambda b, j: (b, j, 0)),
            pl.BlockSpec((1, TN, H), lambda b, j: (b, j, 0)),
        ],
        out_shape=[
            jax.ShapeDtypeStruct((B, 1, H), _F32),
            jax.ShapeDtypeStruct((B, N, H), _F32),
            jax.ShapeDtypeStruct((B, N, H), _F32),
        ],
        compiler_params=pltpu.CompilerParams(
            dimension_semantics=("parallel", "arbitrary")),
    )(question_embedding, qm_b, entity_embedding, W_proj, b_proj, wm0b)


def _hop0_body(rel_ref, tail_ref, me_ref, mo_ref, ent_ref, q_ref,
               wrsq_ref, crs_ref, AX_ref, c2_ref,
               wut_ref, wub_ref, bu_ref, wnext_ref,
               wl_ref, we_ref, wo_ref, ent1_ref, entw1_ref):
    TN, P, D2 = rel_ref.shape[1], rel_ref.shape[2], rel_ref.shape[3]
    H = wut_ref.shape[0]
    relF = rel_ref[0].reshape(TN * P, D2)
    O = jnp.dot(relF, AX_ref[...], preferred_element_type=_F32)
    qdot = jnp.sum(q_ref[0] * wrsq_ref[...]) + crs_ref[0, 0]
    me = me_ref[0][:, :, None]
    mo = mo_ref[0][:, :, None]
    wle = O[:, D2:D2 + 1].reshape(TN, P, 1) + qdot + (1.0 - me) * _NEG
    wlo = O[:, D2 + 1:D2 + 2].reshape(TN, P, 1) + qdot + (1.0 - mo) * _NEG
    wl_ref[0] = jnp.concatenate([wle, wlo], axis=-1).reshape(TN, 2 * P)
    m = jnp.max(jnp.maximum(wle, wlo), axis=1, keepdims=True)
    ee = jnp.exp(wle - m)
    eo = jnp.exp(wlo - m)
    s = jnp.sum(ee + eo, axis=1, keepdims=True)
    we3 = me * (ee / s)
    wo3 = mo * (eo / s)
    we_ref[0] = we3[..., 0]
    wo_ref[0] = wo3[..., 0]

    pre = O[:, :D2] + tail_ref[0].reshape(TN * P, D2) + c2_ref[...]
    msg = _gelu(pre).reshape(TN, P, D2)
    w2 = jnp.concatenate([jnp.broadcast_to(we3, (TN, P, H)),
                          jnp.broadcast_to(wo3, (TN, P, H))], axis=-1)
    agg2 = jnp.sum(msg * w2, axis=1)
    agg = agg2[:, :H] + agg2[:, H:]
    u = (jnp.dot(ent_ref[0], wut_ref[...], preferred_element_type=_F32)
         + jnp.dot(agg, wub_ref[...], preferred_element_type=_F32)
         + bu_ref[...])
    ent1 = _gelu(u)
    ent1_ref[0] = ent1
    entw1_ref[0] = jnp.dot(ent1, wnext_ref[...], preferred_element_type=_F32)


def _hop0(rel2, tail2, mask_e, mask_o, ent0, q,
          wrsq, crs, AX0, c2, wut, wub, bu, wnext, TN):
    B, N, P, D2 = rel2.shape
    H = wut.shape[0]
    NB = N // TN
    NC = AX0.shape[1]
    wfull = lambda *s: pl.BlockSpec(s, lambda b, j: tuple(0 for _ in s))
    return pl.pallas_call(
        _hop0_body,
        grid=(B, NB),
        in_specs=[
            pl.BlockSpec((1, TN, P, D2), lambda b, j: (b, j, 0, 0)),
            pl.BlockSpec((1, TN, P, D2), lambda b, j: (b, j, 0, 0)),
            pl.BlockSpec((1, TN, P), lambda b, j: (b, j, 0)),
            pl.BlockSpec((1, TN, P), lambda b, j: (b, j, 0)),
            pl.BlockSpec((1, TN, H), lambda b, j: (b, j, 0)),
            pl.BlockSpec((1, 1, H), lambda b, j: (b, 0, 0)),
            wfull(1, H), wfull(1, 1), wfull(D2, NC), wfull(1, D2),
            wfull(H, H), wfull(H, H), wfull(1, H), wfull(H, H),
        ],
        out_specs=[
            pl.BlockSpec((1, TN, 2 * P), lambda b, j: (b, j, 0)),
            pl.BlockSpec((1, TN, P), lambda b, j: (b, j, 0)),
            pl.BlockSpec((1, TN, P), lambda b, j: (b, j, 0)),
            pl.BlockSpec((1, TN, H), lambda b, j: (b, j, 0)),
            pl.BlockSpec((1, TN, H), lambda b, j: (b, j, 0)),
        ],
        out_shape=[
            jax.ShapeDtypeStruct((B, N, 2 * P), _F32),
            jax.ShapeDtypeStruct((B, N, P), _F32),
            jax.ShapeDtypeStruct((B, N, P), _F32),
            jax.ShapeDtypeStruct((B, N, H), _F32),
            jax.ShapeDtypeStruct((B, N, H), _F32),
        ],
        compiler_params=pltpu.CompilerParams(
            dimension_semantics=("parallel", "parallel")),
    )(rel2, tail2, mask_e, mask_o, ent0, q,
      wrsq, crs, AX0, c2, wut, wub, bu, wnext)


def _hop1_body(rel_ref, tail_ref, we_ref, wo_ref, ent_ref,
               A2_ref, c2_ref, wut_ref, wub_ref, bu_ref,
               ent2_ref):
    TN, P, D2 = rel_ref.shape[1], rel_ref.shape[2], rel_ref.shape[3]
    H = wut_ref.shape[0]
    relF = rel_ref[0].reshape(TN * P, D2)
    M2 = jnp.dot(relF, A2_ref[...], preferred_element_type=_F32)
    pre = M2 + tail_ref[0].reshape(TN * P, D2) + c2_ref[...]
    msg = _gelu(pre).reshape(TN, P, D2)
    we3 = we_ref[0][:, :, None]
    wo3 = wo_ref[0][:, :, None]
    w2 = jnp.concatenate([jnp.broadcast_to(we3, (TN, P, H)),
                          jnp.broadcast_to(wo3, (TN, P, H))], axis=-1)
    agg2 = jnp.sum(msg * w2, axis=1)
    agg = agg2[:, :H] + agg2[:, H:]
    u = (jnp.dot(ent_ref[0], wut_ref[...], preferred_element_type=_F32)
         + jnp.dot(agg, wub_ref[...], preferred_element_type=_F32)
         + bu_ref[...])
    ent2_ref[0] = _gelu(u)


def _hop1(rel2, tail2, w_e, w_o, ent1, A2, c2, wut, wub, bu, TN):
    B, N, P, D2 = rel2.shape
    H = wut.shape[0]
    NB = N // TN
    wfull = lambda *s: pl.BlockSpec(s, lambda b, j: tuple(0 for _ in s))
    return pl.pallas_call(
        _hop1_body,
        grid=(B, NB),
        in_specs=[
            pl.BlockSpec((1, TN, P, D2), lambda b, j: (b, j, 0, 0)),
            pl.BlockSpec((1, TN, P, D2), lambda b, j: (b, j, 0, 0)),
            pl.BlockSpec((1, TN, P), lambda b, j: (b, j, 0)),
            pl.BlockSpec((1, TN, P), lambda b, j: (b, j, 0)),
            pl.BlockSpec((1, TN, H), lambda b, j: (b, j, 0)),
            wfull(D2, D2), wfull(1, D2), wfull(H, H), wfull(H, H),
            wfull(1, H),
        ],
        out_specs=[pl.BlockSpec((1, TN, H), lambda b, j: (b, j, 0))],
        out_shape=[jax.ShapeDtypeStruct((B, N, H), _F32)],
        compiler_params=pltpu.CompilerParams(
            dimension_semantics=("parallel", "parallel")),
    )(rel2, tail2, w_e, w_o, ent1, A2, c2, wut, wub, bu)[0]


def _epilogue_body(ent2_ref, q_ref, nmask_ref, ws_ref, bs_ref,
                   score_ref, eq_ref):
    ent2 = ent2_ref[0]
    score = (jnp.sum(ent2 * ws_ref[...], axis=-1) + bs_ref[0, 0]
             + (1.0 - nmask_ref[0, 0]) * _NEG)
    score_ref[0, 0] = score
    eq_ref[0, 0] = jnp.sum(ent2 * q_ref[0], axis=-1)


def _epilogue(ent2, q, node_mask, ws_row, bs):
    B, N, H = ent2.shape
    wfull = lambda *s: pl.BlockSpec(s, lambda b: tuple(0 for _ in s))
    score, eq = pl.pallas_call(
        _epilogue_body,
        grid=(B,),
        in_specs=[
            pl.BlockSpec((1, N, H), lambda b: (b, 0, 0)),
            pl.BlockSpec((1, 1, H), lambda b: (b, 0, 0)),
            pl.BlockSpec((1, 1, N), lambda b: (b, 0, 0)),
            wfull(1, H), wfull(1, 1),
        ],
        out_specs=[
            pl.BlockSpec((1, 1, N), lambda b: (b, 0, 0)),
            pl.BlockSpec((1, 1, N), lambda b: (b, 0, 0)),
        ],
        out_shape=[
            jax.ShapeDtypeStruct((B, 1, N), _F32),
            jax.ShapeDtypeStruct((B, 1, N), _F32),
        ],
    )(ent2, q, node_mask.reshape(B, 1, N), ws_row, bs)
    return score.reshape(B, N), eq.reshape(B, N)


def _sc_gather(table, idx, n_batches):
    T, H = table.shape
    E = idx.shape[0]
    per_w = E // _NW
    C = 800
    n_chunks = per_w // C
    n_per_batch = T // n_batches
    w_per_batch = _NW // n_batches
    mesh = plsc.VectorSubcoreMesh(core_axis_name="c", subcore_axis_name="s")

    @functools.partial(
        pl.kernel, mesh=mesh,
        out_type=jax.ShapeDtypeStruct((E, H), _F32),
        scratch_types=[
            pltpu.VMEM((per_w,), jnp.int32),
            pltpu.VMEM((C, H), _F32),
            pltpu.SemaphoreType.DMA,
        ],
        compiler_params=pltpu.CompilerParams(use_tc_tiling_on_sc=False),
    )
    def k(table_hbm, idx_hbm, out_hbm, idx_v, rows_v, sem):
        wid = lax.axis_index("s") * _NC + lax.axis_index("c")
        base = wid * per_w
        boff = (wid // w_per_batch) * n_per_batch
        pltpu.sync_copy(idx_hbm.at[pl.ds(base, per_w)], idx_v)

        def add_off(i):
            sl = pl.ds(i * 16, 16)
            idx_v[sl] = idx_v[sl] + boff
        pl.loop(0, per_w // 16)(add_off)

        def chunk(c):
            pltpu.async_copy(
                table_hbm.at[idx_v.at[pl.ds(c * C, C)]], rows_v, sem).wait()
            pltpu.sync_copy(rows_v, out_hbm.at[pl.ds(base + c * C, C)])
        pl.loop(0, n_chunks)(chunk)

    return k(table, idx)


def _sc_triple(eq_flat, adj_flat, wl_flat, n_batches, n_nodes, log2_r):
    E = adj_flat.shape[0]
    per_w = E // _NW
    C = 2000
    n_chunks = per_w // C
    w_per_batch = _NW // n_batches
    per_batch_e = per_w * w_per_batch
    mesh = plsc.VectorSubcoreMesh(core_axis_name="c", subcore_axis_name="s")

    @functools.partial(
        pl.kernel, mesh=mesh,
        out_type=jax.ShapeDtypeStruct((E,), _F32),
        scratch_types=[
            pltpu.VMEM((n_nodes,), _F32),
            pltpu.VMEM((C,), jnp.int32),
            pltpu.VMEM((C,), _F32),
            pltpu.VMEM((C,), _F32),
        ],
        compiler_params=pltpu.CompilerParams(needs_layout_passes=False),
    )
    def k(eq_hbm, adj_hbm, wl_hbm, out_hbm, eq_v, idx_v, wl_v, out_v):
        wid = lax.axis_index("s") * _NC + lax.axis_index("c")
        b = wid // w_per_batch
        gbase = wid * per_w
        lbase = gbase - b * per_batch_e
        pltpu.sync_copy(eq_hbm.at[pl.ds(b * n_nodes, n_nodes)], eq_v)

        def chunk(c):
            pltpu.sync_copy(adj_hbm.at[pl.ds(gbase + c * C, C)], idx_v)
            pltpu.sync_copy(wl_hbm.at[pl.ds(gbase + c * C, C)], wl_v)

            def step(i):
                sl = pl.ds(i * 16, 16)
                lane_e = lbase + c * C + i * 16 + lax.iota(jnp.int32, 16)
                src = lax.shift_right_logical(lane_e, log2_r)
                vsrc = plsc.load_gather(eq_v, [src])
                vtgt = plsc.load_gather(eq_v, [idx_v[sl]])
                out_v[sl] = vsrc + vtgt + wl_v[sl]
            pl.loop(0, C // 16)(step)
            pltpu.sync_copy(out_v, out_hbm.at[pl.ds(gbase + c * C, C)])
        pl.loop(0, n_chunks)(chunk)

    return k(eq_flat, adj_flat, wl_flat)


def kernel(question_embedding, question_mask, entity_embedding, rel_embedding,
           adj, node_mask, adj_mask,
           W_proj, b_proj, W_relp, b_relp,
           W_msg0, b_msg0, W_msg1, b_msg1,
           W_upd0, b_upd0, W_upd1, b_upd1,
           W_rs, b_rs, W_s, b_s):
    B, LQ, E = question_embedding.shape
    _, N, R, RD = rel_embedding.shape
    H = W_proj.shape[1]
    TN = 400
    log2_r = R.bit_length() - 1
    assert (1 << log2_r) == R

    P = R // 2
    D2 = 2 * RD
    A0 = W_relp @ W_msg0[:H]
    c0 = b_relp @ W_msg0[:H] + b_msg0
    A1 = W_relp @ W_msg1[:H]
    c1 = b_relp @ W_msg1[:H] + b_msg1
    ars = W_relp @ W_rs[:H, 0]
    crs = (b_relp @ W_rs[:H, 0] + b_rs[0]).reshape(1, 1)
    wrsq = W_rs[H:, 0][None]
    wm0b = W_msg0[H:]
    wm1b = W_msg1[H:]
    bproj = b_proj[None]
    bu0 = b_upd0[None]
    bu1 = b_upd1[None]
    ws_row = W_s[:, 0][None]
    bs = b_s.reshape(1, 1)
    qm_b = jnp.broadcast_to(question_mask[:, :, None], (B, LQ, H))

    z = jnp.zeros((RD, H), _F32)
    A2_0 = jnp.block([[A0, z], [z, A0]])
    A2_1 = jnp.block([[A1, z], [z, A1]])
    zc = jnp.zeros((RD,), _F32)
    AX0 = jnp.concatenate(
        [A2_0, jnp.concatenate([ars, zc])[:, None],
         jnp.concatenate([zc, ars])[:, None]], axis=1)
    c2_0 = jnp.concatenate([c0, c0])[None]
    c2_1 = jnp.concatenate([c1, c1])[None]

    rel2 = rel_embedding.reshape(B, N, P, D2)
    mask_e = adj_mask[:, :, 0::2]
    mask_o = adj_mask[:, :, 1::2]
    adj_flat = adj.reshape(B * N * R)

    q, ent0, entw0 = _prologue(question_embedding, qm_b, entity_embedding,
                               W_proj, bproj, wm0b, TN)
    tail0 = _sc_gather(entw0.reshape(B * N, H), adj_flat, B)
    wl, w_e, w_o, ent1, entw1 = _hop0(
        rel2, tail0.reshape(B, N, P, D2), mask_e, mask_o, ent0, q,
        wrsq, crs, AX0, c2_0, W_upd0[:H], W_upd0[H:], bu0, wm1b, TN)
    tail1 = _sc_gather(entw1.reshape(B * N, H), adj_flat, B)
    ent2 = _hop1(rel2, tail1.reshape(B, N, P, D2), w_e, w_o, ent1,
                 A2_1, c2_1, W_upd1[:H], W_upd1[H:], bu1, TN)
    score, eq = _epilogue(ent2, q, node_mask, ws_row, bs)
    ts = _sc_triple(eq.reshape(B * N), adj_flat, wl.reshape(B * N * R),
                    B, N, log2_r)
    return (ent2, score, wl, ts.reshape(B, N * R))


# speedup vs baseline: 1.2243x; 1.0052x over previous
import functools

import jax
import jax.numpy as jnp
from jax import lax
from jax.experimental import pallas as pl
from jax.experimental.pallas import tpu as pltpu
from jax.experimental.pallas import tpu_sc as plsc

_NEG = float(jnp.finfo(jnp.float32).min)
_F32 = jnp.float32

_NC, _NS = 2, 16
_NW = _NC * _NS


def _gelu(x):
    return 0.5 * x * (1.0 + lax.erf(x * (2.0 ** -0.5)))


def _prologue_body(qe_ref, qm_ref, ee_ref, wproj_ref, bproj_ref, wm0b_ref,
                   q_ref, ent0_ref, entw0_ref):
    j = pl.program_id(1)
    ent0 = (jnp.dot(ee_ref[0], wproj_ref[...], preferred_element_type=_F32)
            + bproj_ref[...])
    ent0_ref[0] = ent0
    entw0_ref[0] = jnp.dot(ent0, wm0b_ref[...], preferred_element_type=_F32)

    @pl.when(j == 0)
    def _():
        qe = (jnp.dot(qe_ref[0], wproj_ref[...], preferred_element_type=_F32)
              + bproj_ref[...])
        qm = qm_ref[0]
        num = jnp.sum(qe * qm, axis=0)
        den = jnp.sum(qm, axis=0)
        q_ref[0, 0] = num / den


def _prologue(question_embedding, qm_b, entity_embedding,
              W_proj, b_proj, wm0b, TN):
    B, LQ, E = question_embedding.shape
    _, N, _ = entity_embedding.shape
    H = W_proj.shape[1]
    NB = N // TN
    return pl.pallas_call(
        _prologue_body,
        grid=(B, NB),
        in_specs=[
            pl.BlockSpec((1, LQ, E), lambda b, j: (b, 0, 0)),
            pl.BlockSpec((1, LQ, H), lambda b, j: (b, 0, 0)),
            pl.BlockSpec((1, TN, E), lambda b, j: (b, j, 0)),
            pl.BlockSpec((E, H), lambda b, j: (0, 0)),
            pl.BlockSpec((1, H), lambda b, j: (0, 0)),
            pl.BlockSpec((H, H), lambda b, j: (0, 0)),
        ],
        out_specs=[
            pl.BlockSpec((1, 1, H), lambda b, j: (b, 0, 0)),
            pl.BlockSpec((1, TN, H), lambda b, j: (b, j, 0)),
            pl.BlockSpec((1, TN, H), lambda b, j: (b, j, 0)),
        ],
        out_shape=[
            jax.ShapeDtypeStruct((B, 1, H), _F32),
            jax.ShapeDtypeStruct((B, N, H), _F32),
            jax.ShapeDtypeStruct((B, N, H), _F32),
        ],
        compiler_params=pltpu.CompilerParams(
            dimension_semantics=("parallel", "arbitrary")),
    )(question_embedding, qm_b, entity_embedding, W_proj, b_proj, wm0b)


def _hop0_body(rel_ref, tail_ref, me_ref, mo_ref, ent_ref, q_ref,
               wrsq_ref, crs_ref, AX_ref, c2_ref,
               wut_ref, wub_ref, bu_ref, wnext_ref,
               wl_ref, we_ref, wo_ref, ent1_ref, entw1_ref):
    TN, P, D2 = rel_ref.shape[1], rel_ref.shape[2], rel_ref.shape[3]
    H = wut_ref.shape[0]
    relF = rel_ref[0].reshape(TN * P, D2)
    O = jnp.dot(relF, AX_ref[...], preferred_element_type=_F32)
    qdot = jnp.sum(q_ref[0] * wrsq_ref[...]) + crs_ref[0, 0]
    me = me_ref[0][:, :, None]
    mo = mo_ref[0][:, :, None]
    wle = O[:, D2:D2 + 1].reshape(TN, P, 1) + qdot + (1.0 - me) * _NEG
    wlo = O[:, D2 + 1:D2 + 2].reshape(TN, P, 1) + qdot + (1.0 - mo) * _NEG
    wl_ref[0] = jnp.concatenate([wle, wlo], axis=-1).reshape(TN, 2 * P)
    m = jnp.max(jnp.maximum(wle, wlo), axis=1, keepdims=True)
    ee = jnp.exp(wle - m)
    eo = jnp.exp(wlo - m)
    s = jnp.sum(ee + eo, axis=1, keepdims=True)
    we3 = me * (ee / s)
    wo3 = mo * (eo / s)
    we_ref[0] = we3[..., 0]
    wo_ref[0] = wo3[..., 0]

    pre = O[:, :D2] + tail_ref[0].reshape(TN * P, D2) + c2_ref[...]
    msg = _gelu(pre).reshape(TN, P, D2)
    w2 = jnp.concatenate([jnp.broadcast_to(we3, (TN, P, H)),
                          jnp.broadcast_to(wo3, (TN, P, H))], axis=-1)
    agg2 = jnp.sum(msg * w2, axis=1)
    agg = agg2[:, :H] + agg2[:, H:]
    u = (jnp.dot(ent_ref[0], wut_ref[...], preferred_element_type=_F32)
         + jnp.dot(agg, wub_ref[...], preferred_element_type=_F32)
         + bu_ref[...])
    ent1 = _gelu(u)
    ent1_ref[0] = ent1
    entw1_ref[0] = jnp.dot(ent1, wnext_ref[...], preferred_element_type=_F32)


def _hop0(rel2, tail2, mask_e, mask_o, ent0, q,
          wrsq, crs, AX0, c2, wut, wub, bu, wnext, TN):
    B, N, P, D2 = rel2.shape
    H = wut.shape[0]
    NB = N // TN
    NC = AX0.shape[1]
    wfull = lambda *s: pl.BlockSpec(s, lambda b, j: tuple(0 for _ in s))
    return pl.pallas_call(
        _hop0_body,
        grid=(B, NB),
        in_specs=[
            pl.BlockSpec((1, TN, P, D2), lambda b, j: (b, j, 0, 0)),
            pl.BlockSpec((1, TN, P, D2), lambda b, j: (b, j, 0, 0)),
            pl.BlockSpec((1, TN, P), lambda b, j: (b, j, 0)),
            pl.BlockSpec((1, TN, P), lambda b, j: (b, j, 0)),
            pl.BlockSpec((1, TN, H), lambda b, j: (b, j, 0)),
            pl.BlockSpec((1, 1, H), lambda b, j: (b, 0, 0)),
            wfull(1, H), wfull(1, 1), wfull(D2, NC), wfull(1, D2),
            wfull(H, H), wfull(H, H), wfull(1, H), wfull(H, H),
        ],
        out_specs=[
            pl.BlockSpec((1, TN, 2 * P), lambda b, j: (b, j, 0)),
            pl.BlockSpec((1, TN, P), lambda b, j: (b, j, 0)),
            pl.BlockSpec((1, TN, P), lambda b, j: (b, j, 0)),
            pl.BlockSpec((1, TN, H), lambda b, j: (b, j, 0)),
            pl.BlockSpec((1, TN, H), lambda b, j: (b, j, 0)),
        ],
        out_shape=[
            jax.ShapeDtypeStruct((B, N, 2 * P), _F32),
            jax.ShapeDtypeStruct((B, N, P), _F32),
            jax.ShapeDtypeStruct((B, N, P), _F32),
            jax.ShapeDtypeStruct((B, N, H), _F32),
            jax.ShapeDtypeStruct((B, N, H), _F32),
        ],
        compiler_params=pltpu.CompilerParams(
            dimension_semantics=("parallel", "parallel")),
    )(rel2, tail2, mask_e, mask_o, ent0, q,
      wrsq, crs, AX0, c2, wut, wub, bu, wnext)


def _hop1_body(rel_ref, tail_ref, we_ref, wo_ref, ent_ref,
               A2_ref, c2_ref, wut_ref, wub_ref, bu_ref,
               ent2_ref):
    TN, P, D2 = rel_ref.shape[1], rel_ref.shape[2], rel_ref.shape[3]
    H = wut_ref.shape[0]
    relF = rel_ref[0].reshape(TN * P, D2)
    M2 = jnp.dot(relF, A2_ref[...], preferred_element_type=_F32)
    pre = M2 + tail_ref[0].reshape(TN * P, D2) + c2_ref[...]
    msg = _gelu(pre).reshape(TN, P, D2)
    we3 = we_ref[0][:, :, None]
    wo3 = wo_ref[0][:, :, None]
    w2 = jnp.concatenate([jnp.broadcast_to(we3, (TN, P, H)),
                          jnp.broadcast_to(wo3, (TN, P, H))], axis=-1)
    agg2 = jnp.sum(msg * w2, axis=1)
    agg = agg2[:, :H] + agg2[:, H:]
    u = (jnp.dot(ent_ref[0], wut_ref[...], preferred_element_type=_F32)
         + jnp.dot(agg, wub_ref[...], preferred_element_type=_F32)
         + bu_ref[...])
    ent2_ref[0] = _gelu(u)


def _hop1(rel2, tail2, w_e, w_o, ent1, A2, c2, wut, wub, bu, TN):
    B, N, P, D2 = rel2.shape
    H = wut.shape[0]
    NB = N // TN
    wfull = lambda *s: pl.BlockSpec(s, lambda b, j: tuple(0 for _ in s))
    return pl.pallas_call(
        _hop1_body,
        grid=(B, NB),
        in_specs=[
            pl.BlockSpec((1, TN, P, D2), lambda b, j: (b, j, 0, 0)),
            pl.BlockSpec((1, TN, P, D2), lambda b, j: (b, j, 0, 0)),
            pl.BlockSpec((1, TN, P), lambda b, j: (b, j, 0)),
            pl.BlockSpec((1, TN, P), lambda b, j: (b, j, 0)),
            pl.BlockSpec((1, TN, H), lambda b, j: (b, j, 0)),
            wfull(D2, D2), wfull(1, D2), wfull(H, H), wfull(H, H),
            wfull(1, H),
        ],
        out_specs=[pl.BlockSpec((1, TN, H), lambda b, j: (b, j, 0))],
        out_shape=[jax.ShapeDtypeStruct((B, N, H), _F32)],
        compiler_params=pltpu.CompilerParams(
            dimension_semantics=("parallel", "parallel")),
    )(rel2, tail2, w_e, w_o, ent1, A2, c2, wut, wub, bu)[0]


def _epilogue_body(ent2_ref, q_ref, nmask_ref, ws_ref, bs_ref,
                   score_ref, eq_ref):
    ent2 = ent2_ref[0]
    score = (jnp.sum(ent2 * ws_ref[...], axis=-1) + bs_ref[0, 0]
             + (1.0 - nmask_ref[0, 0]) * _NEG)
    score_ref[0, 0] = score
    eq_ref[0, 0] = jnp.sum(ent2 * q_ref[0], axis=-1)


def _epilogue(ent2, q, node_mask, ws_row, bs):
    B, N, H = ent2.shape
    wfull = lambda *s: pl.BlockSpec(s, lambda b: tuple(0 for _ in s))
    score, eq = pl.pallas_call(
        _epilogue_body,
        grid=(B,),
        in_specs=[
            pl.BlockSpec((1, N, H), lambda b: (b, 0, 0)),
            pl.BlockSpec((1, 1, H), lambda b: (b, 0, 0)),
            pl.BlockSpec((1, 1, N), lambda b: (b, 0, 0)),
            wfull(1, H), wfull(1, 1),
        ],
        out_specs=[
            pl.BlockSpec((1, 1, N), lambda b: (b, 0, 0)),
            pl.BlockSpec((1, 1, N), lambda b: (b, 0, 0)),
        ],
        out_shape=[
            jax.ShapeDtypeStruct((B, 1, N), _F32),
            jax.ShapeDtypeStruct((B, 1, N), _F32),
        ],
    )(ent2, q, node_mask.reshape(B, 1, N), ws_row, bs)
    return score.reshape(B, N), eq.reshape(B, N)


def _sc_gather(table, idx, n_batches):
    T, H = table.shape
    E = idx.shape[0]
    per_w = E // _NW
    C = 400
    NBUF = 4
    n_chunks = per_w // C
    n_per_batch = T // n_batches
    w_per_batch = _NW // n_batches
    mesh = plsc.VectorSubcoreMesh(core_axis_name="c", subcore_axis_name="s")

    @functools.partial(
        pl.kernel, mesh=mesh,
        out_type=jax.ShapeDtypeStruct((E, H), _F32),
        scratch_types=(
            [pltpu.VMEM((per_w,), jnp.int32)]
            + [pltpu.VMEM((C, H), _F32) for _ in range(NBUF)]
            + [pltpu.SemaphoreType.DMA for _ in range(2 * NBUF)]
        ),
        compiler_params=pltpu.CompilerParams(use_tc_tiling_on_sc=False),
    )
    def k(table_hbm, idx_hbm, out_hbm, idx_v, *bufs_sems):
        bufs = bufs_sems[:NBUF]
        gs = bufs_sems[NBUF:2 * NBUF]
        ws = bufs_sems[2 * NBUF:]
        wid = lax.axis_index("s") * _NC + lax.axis_index("c")
        base = wid * per_w
        boff = (wid // w_per_batch) * n_per_batch
        pltpu.sync_copy(idx_hbm.at[pl.ds(base, per_w)], idx_v)

        def add_off(i):
            sl = pl.ds(i * 16, 16)
            idx_v[sl] = idx_v[sl] + boff
        pl.loop(0, per_w // 16)(add_off)

        def gather(c, k_):
            pltpu.async_copy(
                table_hbm.at[idx_v.at[pl.ds(c * C, C)]], bufs[k_], gs[k_])

        def gather_wait(k_):
            pltpu.make_async_copy(
                table_hbm.at[idx_v.at[pl.ds(0, C)]], bufs[k_], gs[k_]).wait()

        def write(c, k_):
            pltpu.async_copy(bufs[k_], out_hbm.at[pl.ds(base + c * C, C)],
                             ws[k_])

        def write_wait(k_):
            pltpu.make_async_copy(
                bufs[k_], out_hbm.at[pl.ds(base, C)], ws[k_]).wait()

        for k_ in range(NBUF):
            gather(k_, k_)

        def round_(c):
            for k_ in range(NBUF):
                cc = c + k_
                gather_wait(k_)
                write(cc, k_)

                @pl.when(cc + NBUF < n_chunks)
                def _():
                    write_wait(k_)
                    gather(cc + NBUF, k_)
        pl.loop(0, n_chunks, step=NBUF)(round_)

        for k_ in range(NBUF):
            write_wait(k_)

    return k(table, idx)


def _sc_triple(eq_flat, adj_flat, wl_flat, n_batches, n_nodes, log2_r):
    E = adj_flat.shape[0]
    per_w = E // _NW
    C = 2000
    n_chunks = per_w // C
    w_per_batch = _NW // n_batches
    per_batch_e = per_w * w_per_batch
    mesh = plsc.VectorSubcoreMesh(core_axis_name="c", subcore_axis_name="s")

    @functools.partial(
        pl.kernel, mesh=mesh,
        out_type=jax.ShapeDtypeStruct((E,), _F32),
        scratch_types=[
            pltpu.VMEM((n_nodes,), _F32),
            pltpu.VMEM((C,), jnp.int32),
            pltpu.VMEM((C,), _F32),
            pltpu.VMEM((C,), _F32),
        ],
        compiler_params=pltpu.CompilerParams(needs_layout_passes=False),
    )
    def k(eq_hbm, adj_hbm, wl_hbm, out_hbm, eq_v, idx_v, wl_v, out_v):
        wid = lax.axis_index("s") * _NC + lax.axis_index("c")
        b = wid // w_per_batch
        gbase = wid * per_w
        lbase = gbase - b * per_batch_e
        pltpu.sync_copy(eq_hbm.at[pl.ds(b * n_nodes, n_nodes)], eq_v)

        def chunk(c):
            pltpu.sync_copy(adj_hbm.at[pl.ds(gbase + c * C, C)], idx_v)
            pltpu.sync_copy(wl_hbm.at[pl.ds(gbase + c * C, C)], wl_v)

            def step(i):
                sl = pl.ds(i * 16, 16)
                lane_e = lbase + c * C + i * 16 + lax.iota(jnp.int32, 16)
                src = lax.shift_right_logical(lane_e, log2_r)
                vsrc = plsc.load_gather(eq_v, [src])
                vtgt = plsc.load_gather(eq_v, [idx_v[sl]])
                out_v[sl] = vsrc + vtgt + wl_v[sl]
            pl.loop(0, C // 16)(step)
            pltpu.sync_copy(out_v, out_hbm.at[pl.ds(gbase + c * C, C)])
        pl.loop(0, n_chunks)(chunk)

    return k(eq_flat, adj_flat, wl_flat)


def kernel(question_embedding, question_mask, entity_embedding, rel_embedding,
           adj, node_mask, adj_mask,
           W_proj, b_proj, W_relp, b_relp,
           W_msg0, b_msg0, W_msg1, b_msg1,
           W_upd0, b_upd0, W_upd1, b_upd1,
           W_rs, b_rs, W_s, b_s):
    B, LQ, E = question_embedding.shape
    _, N, R, RD = rel_embedding.shape
    H = W_proj.shape[1]
    TN = 400
    log2_r = R.bit_length() - 1
    assert (1 << log2_r) == R

    P = R // 2
    D2 = 2 * RD
    A0 = W_relp @ W_msg0[:H]
    c0 = b_relp @ W_msg0[:H] + b_msg0
    A1 = W_relp @ W_msg1[:H]
    c1 = b_relp @ W_msg1[:H] + b_msg1
    ars = W_relp @ W_rs[:H, 0]
    crs = (b_relp @ W_rs[:H, 0] + b_rs[0]).reshape(1, 1)
    wrsq = W_rs[H:, 0][None]
    wm0b = W_msg0[H:]
    wm1b = W_msg1[H:]
    bproj = b_proj[None]
    bu0 = b_upd0[None]
    bu1 = b_upd1[None]
    ws_row = W_s[:, 0][None]
    bs = b_s.reshape(1, 1)
    qm_b = jnp.broadcast_to(question_mask[:, :, None], (B, LQ, H))

    z = jnp.zeros((RD, H), _F32)
    A2_0 = jnp.block([[A0, z], [z, A0]])
    A2_1 = jnp.block([[A1, z], [z, A1]])
    zc = jnp.zeros((RD,), _F32)
    AX0 = jnp.concatenate(
        [A2_0, jnp.concatenate([ars, zc])[:, None],
         jnp.concatenate([zc, ars])[:, None]], axis=1)
    c2_0 = jnp.concatenate([c0, c0])[None]
    c2_1 = jnp.concatenate([c1, c1])[None]

    rel2 = rel_embedding.reshape(B, N, P, D2)
    mask_e = adj_mask[:, :, 0::2]
    mask_o = adj_mask[:, :, 1::2]
    adj_flat = adj.reshape(B * N * R)

    q, ent0, entw0 = _prologue(question_embedding, qm_b, entity_embedding,
                               W_proj, bproj, wm0b, TN)
    tail0 = _sc_gather(entw0.reshape(B * N, H), adj_flat, B)
    wl, w_e, w_o, ent1, entw1 = _hop0(
        rel2, tail0.reshape(B, N, P, D2), mask_e, mask_o, ent0, q,
        wrsq, crs, AX0, c2_0, W_upd0[:H], W_upd0[H:], bu0, wm1b, TN)
    tail1 = _sc_gather(entw1.reshape(B * N, H), adj_flat, B)
    ent2 = _hop1(rel2, tail1.reshape(B, N, P, D2), w_e, w_o, ent1,
                 A2_1, c2_1, W_upd1[:H], W_upd1[H:], bu1, TN)
    score, eq = _epilogue(ent2, q, node_mask, ws_row, bs)
    ts = _sc_triple(eq.reshape(B * N), adj_flat, wl.reshape(B * N * R),
                    B, N, log2_r)
    return (ent2, score, wl, ts.reshape(B, N * R))
